# Initial kernel scaffold; baseline (speedup 1.0000x reference)
import jax
import jax.numpy as jnp
from jax.experimental import pallas as pl


def kernel(x, edge_indices, W1, b1, W2, b2):
    raise NotImplementedError("write your pallas kernel here")


# speedup vs baseline: 3.8527x; 3.8527x over previous
import functools

import jax
import jax.numpy as jnp
from jax import lax
from jax.experimental import pallas as pl
from jax.experimental.pallas import tpu as pltpu
from jax.experimental.pallas import tpu_sc as plsc

_N = 10000
_E = 160000
_D = 256
_EMB = 512
_NADJ = 2
_NC = 2
_NS = 16
_H = 128
_ET = _E // _NS
_C = 80
_CH = _ET // _C
_P = 5
_PC = _CH // _P
_RT = _N // _NS
_ZR = 25
_ZD = 125

_sc_mesh = plsc.VectorSubcoreMesh(
    core_axis_name="c", subcore_axis_name="s", num_cores=_NC, num_subcores=_NS
)


@functools.partial(
    pl.kernel,
    out_type=(
        jax.ShapeDtypeStruct((_NADJ, _NC, _NS, _RT, _H), jnp.float32),
        jax.ShapeDtypeStruct((_NADJ, _NS, _RT, 16), jnp.float32),
    ),
    mesh=_sc_mesh,
    scratch_types=[
        pltpu.VMEM_SHARED((_N, _H), jnp.float32),
        pltpu.VMEM_SHARED((_N, 16), jnp.float32),
        pltpu.VMEM((_PC, _C), jnp.int32),
        pltpu.VMEM((_PC, _C), jnp.int32),
        pltpu.VMEM((_C, _H), jnp.float32),
        pltpu.VMEM((_C, 16), jnp.float32),
        pltpu.VMEM((_ZR, _H), jnp.float32),
        pltpu.VMEM((_ZD, 16), jnp.float32),
        pltpu.SemaphoreType.DMA,
    ],
    compiler_params=pltpu.CompilerParams(use_tc_tiling_on_sc=False),
)
def _agg(x2_hbm, er_hbm, sums_hbm, degw_hbm, acc_sh, deg_sh, src_v, dst_v,
         rows_v, ones_v, zrow_v, zdeg_v, sem):
    c = lax.axis_index("c")
    s = lax.axis_index("s")
    c_n = c * _N
    row0 = s * _RT

    ones16 = jnp.ones((16,), jnp.float32)
    zeros16 = jnp.zeros((16,), jnp.float32)

    def init_ones(i, carry):
        ones_v[i, :] = ones16
        return carry

    lax.fori_loop(0, _C, init_ones, 0)

    def init_zrow(i, carry):
        for t in range(_H // 16):
            zrow_v[i, pl.ds(t * 16, 16)] = zeros16
        return carry

    lax.fori_loop(0, _ZR, init_zrow, 0)

    def init_zdeg(i, carry):
        zdeg_v[i, :] = zeros16
        return carry

    lax.fori_loop(0, _ZD, init_zdeg, 0)

    for a in range(_NADJ):
        def zero_acc(z, carry):
            pltpu.sync_copy(zrow_v, acc_sh.at[pl.ds(row0 + z * _ZR, _ZR)])
            return carry

        lax.fori_loop(0, _RT // _ZR, zero_acc, 0)
        for z in range(_RT // _ZD):
            pltpu.sync_copy(zdeg_v, deg_sh.at[pl.ds(row0 + z * _ZD, _ZD)])
        plsc.subcore_barrier()

        for p in range(_P):
            pltpu.sync_copy(er_hbm.at[a, 0, s, p], src_v)
            pltpu.sync_copy(er_hbm.at[a, 1, s, p], dst_v)

            def adjust(ch, carry):
                for t in range(_C // 16):
                    sl = pl.ds(t * 16, 16)
                    src_v[ch, sl] = src_v[ch, sl] + c_n
                return carry

            lax.fori_loop(0, _PC, adjust, 0)

            def chunk(j, carry):
                pltpu.async_copy(x2_hbm.at[src_v.at[j]], rows_v, sem).wait()
                pltpu.sync_copy(rows_v, acc_sh.at[dst_v.at[j]], add=True)

                @pl.when(c == a)
                def _():
                    pltpu.sync_copy(ones_v, deg_sh.at[dst_v.at[j]], add=True)

                return carry

            lax.fori_loop(0, _PC, chunk, 0)
        plsc.subcore_barrier()

        pltpu.sync_copy(acc_sh.at[pl.ds(row0, _RT)], sums_hbm.at[a, c, s])

        @pl.when(c == a)
        def _():
            pltpu.sync_copy(deg_sh.at[pl.ds(row0, _RT)], degw_hbm.at[a, s])


_BN = 1000


def _mlp_body(x_ref, s00_ref, s01_ref, s10_ref, s11_ref, d0_ref, d1_ref,
              w1x_ref, w1a_ref, w1b_ref, w1c_ref, w1d_ref, b1_ref,
              w2_ref, b2_ref, out_ref):
    r0 = 1.0 / jnp.maximum(d0_ref[:, 0:1], 1.0)
    r1 = 1.0 / jnp.maximum(d1_ref[:, 0:1], 1.0)
    dot = functools.partial(jnp.dot, precision=lax.Precision.HIGHEST,
                            preferred_element_type=jnp.float32)
    acc = dot(x_ref[...], w1x_ref[...])
    acc = acc + dot(s00_ref[...] * r0, w1a_ref[...])
    acc = acc + dot(s01_ref[...] * r0, w1b_ref[...])
    acc = acc + dot(s10_ref[...] * r1, w1c_ref[...])
    acc = acc + dot(s11_ref[...] * r1, w1d_ref[...])
    h = jnp.tanh(acc + b1_ref[...])
    out_ref[...] = dot(h, w2_ref[...]) + b2_ref[...]


def _mlp(x, s00, s01, s10, s11, d0, d1, w1x, w1a, w1b, w1c, w1d, b1, w2, b2):
    rows = lambda w: pl.BlockSpec((_BN, w), lambda i: (i, 0))
    whole = lambda r, w: pl.BlockSpec((r, w), lambda i: (0, 0))
    return pl.pallas_call(
        _mlp_body,
        grid=(_N // _BN,),
        in_specs=[
            rows(_D),
            rows(_H), rows(_H), rows(_H), rows(_H),
            rows(16), rows(16),
            whole(_D, _D),
            whole(_H, _D), whole(_H, _D), whole(_H, _D), whole(_H, _D),
            whole(1, _D),
            whole(_D, _EMB),
            whole(1, _EMB),
        ],
        out_specs=rows(_EMB),
        out_shape=jax.ShapeDtypeStruct((_N, _EMB), jnp.float32),
    )(x, s00, s01, s10, s11, d0, d1, w1x, w1a, w1b, w1c, w1d, b1, w2, b2)


def kernel(x, edge_indices, W1, b1, W2, b2):
    x2 = jnp.concatenate([x[:, :_H], x[:, _H:]], axis=0)
    er = edge_indices.reshape(_NADJ, 2, _NS, _P, _PC, _C)
    sums, degw = _agg(x2, er)
    sums = sums.reshape(_NADJ, _NC, _N, _H)
    degw = degw.reshape(_NADJ, _N, 16)
    out = _mlp(
        x,
        sums[0, 0], sums[0, 1], sums[1, 0], sums[1, 1],
        degw[0], degw[1],
        W1[:_D], W1[_D:_D + _H], W1[_D + _H:2 * _D],
        W1[2 * _D:2 * _D + _H], W1[2 * _D + _H:],
        b1.reshape(1, _D), W2, b2.reshape(1, _EMB),
    )
    return out


# speedup vs baseline: 5.0283x; 1.3051x over previous
import functools

import jax
import jax.numpy as jnp
from jax import lax
from jax.experimental import pallas as pl
from jax.experimental.pallas import tpu as pltpu
from jax.experimental.pallas import tpu_sc as plsc

_N = 10000
_E = 160000
_D = 256
_EMB = 512
_NADJ = 2
_NC = 2
_NS = 16
_H = 128
_ET = _E // _NS
_C = 80
_CH = _ET // _C
_P = 5
_PC = _CH // _P
_RT = _N // _NS
_ZR = 25
_ZD = 125

_sc_mesh = plsc.VectorSubcoreMesh(
    core_axis_name="c", subcore_axis_name="s", num_cores=_NC, num_subcores=_NS
)


@functools.partial(
    pl.kernel,
    out_type=(
        jax.ShapeDtypeStruct((_NADJ, _NC, _NS, _RT, _H), jnp.float32),
        jax.ShapeDtypeStruct((_NADJ, _NS, _RT, 16), jnp.float32),
    ),
    mesh=_sc_mesh,
    scratch_types=[
        pltpu.VMEM_SHARED((_N, _H), jnp.float32),
        pltpu.VMEM_SHARED((_N, 16), jnp.float32),
        pltpu.VMEM((_PC, _C), jnp.int32),
        pltpu.VMEM((_PC, _C), jnp.int32),
        pltpu.VMEM((_C, _H), jnp.float32),
        pltpu.VMEM((_C, _H), jnp.float32),
        pltpu.VMEM((_C, 16), jnp.float32),
        pltpu.VMEM((_ZR, _H), jnp.float32),
        pltpu.VMEM((_ZD, 16), jnp.float32),
        pltpu.SemaphoreType.DMA,
        pltpu.SemaphoreType.DMA,
        pltpu.SemaphoreType.DMA,
    ],
    compiler_params=pltpu.CompilerParams(use_tc_tiling_on_sc=False),
)
def _agg(x2_hbm, er_hbm, sums_hbm, degw_hbm, acc_sh, deg_sh, src_v, dst_v,
         rows0_v, rows1_v, ones_v, zrow_v, zdeg_v, semg0, semg1, semd):
    c = lax.axis_index("c")
    s = lax.axis_index("s")
    row0 = s * _RT

    ones16 = jnp.ones((16,), jnp.float32)
    zeros16 = jnp.zeros((16,), jnp.float32)

    def init_ones(i, carry):
        ones_v[i, :] = ones16
        return carry

    lax.fori_loop(0, _C, init_ones, 0)

    def init_zrow(i, carry):
        for t in range(_H // 16):
            zrow_v[i, pl.ds(t * 16, 16)] = zeros16
        return carry

    lax.fori_loop(0, _ZR, init_zrow, 0)

    def init_zdeg(i, carry):
        zdeg_v[i, :] = zeros16
        return carry

    lax.fori_loop(0, _ZD, init_zdeg, 0)

    def g_start(j, buf, sem):
        pltpu.async_copy(x2_hbm.at[src_v.at[j]], buf, sem)

    def g_wait(buf, sem):
        pltpu.make_async_copy(x2_hbm.at[src_v.at[0]], buf, sem).wait()

    def stage(a, p):
        pltpu.sync_copy(er_hbm.at[a, 0, s, p], src_v)
        pltpu.sync_copy(er_hbm.at[a, 1, s, p], dst_v)

        def adjust(ch, carry):
            for t in range(_C // 16):
                sl = pl.ds(t * 16, 16)
                v = src_v[ch, sl]
                src_v[ch, sl] = v + v + c
            return carry

        lax.fori_loop(0, _PC, adjust, 0)

    def run_pass(a):
        do_deg = c == a

        def scatter(j, buf):
            @pl.when(do_deg)
            def _():
                pltpu.async_copy(ones_v, deg_sh.at[dst_v.at[j]], semd, add=True)

            pltpu.sync_copy(buf, acc_sh.at[dst_v.at[j]], add=True)

            @pl.when(do_deg)
            def _():
                pltpu.make_async_copy(ones_v, deg_sh.at[dst_v.at[0]], semd).wait()

        g_start(0, rows0_v, semg0)

        def pair(k, carry):
            j0 = 2 * k
            g_wait(rows0_v, semg0)
            g_start(j0 + 1, rows1_v, semg1)
            scatter(j0, rows0_v)
            g_wait(rows1_v, semg1)
            g_start(j0 + 2, rows0_v, semg0)
            scatter(j0 + 1, rows1_v)
            return carry

        lax.fori_loop(0, (_PC - 1) // 2, pair, 0)
        g_wait(rows0_v, semg0)
        scatter(_PC - 1, rows0_v)

    for a in range(_NADJ):
        def zero_acc(z, carry):
            pltpu.sync_copy(zrow_v, acc_sh.at[pl.ds(row0 + z * _ZR, _ZR)])
            return carry

        lax.fori_loop(0, _RT // _ZR, zero_acc, 0)
        for z in range(_RT // _ZD):
            pltpu.sync_copy(zdeg_v, deg_sh.at[pl.ds(row0 + z * _ZD, _ZD)])
        stage(a, 0)
        plsc.subcore_barrier()

        run_pass(a)
        for p in range(1, _P):
            stage(a, p)
            run_pass(a)
        plsc.subcore_barrier()

        pltpu.sync_copy(acc_sh.at[pl.ds(row0, _RT)], sums_hbm.at[a, c, s])

        @pl.when(c == a)
        def _():
            pltpu.sync_copy(deg_sh.at[pl.ds(row0, _RT)], degw_hbm.at[a, s])


_BN = 1000


def _mlp_body(x_ref, s00_ref, s01_ref, s10_ref, s11_ref, d0_ref, d1_ref,
              w1x_ref, w1a_ref, w1b_ref, w1c_ref, w1d_ref, b1_ref,
              w2_ref, b2_ref, out_ref):
    r0 = 1.0 / jnp.maximum(d0_ref[0, :, 0:1], 1.0)
    r1 = 1.0 / jnp.maximum(d1_ref[0, :, 0:1], 1.0)
    dot = functools.partial(jnp.dot, precision=lax.Precision.HIGHEST,
                            preferred_element_type=jnp.float32)
    acc = dot(x_ref[...], w1x_ref[...])
    acc = acc + dot(s00_ref[0, 0] * r0, w1a_ref[...])
    acc = acc + dot(s01_ref[0, 0] * r0, w1b_ref[...])
    acc = acc + dot(s10_ref[0, 0] * r1, w1c_ref[...])
    acc = acc + dot(s11_ref[0, 0] * r1, w1d_ref[...])
    h = jnp.tanh(acc + b1_ref[...])
    out_ref[...] = dot(h, w2_ref[...]) + b2_ref[...]


def _mlp(x, sums, degw, W1, b1, W2, b2):
    sblk = lambda a, c: pl.BlockSpec((1, 1, _BN, _H), lambda i, a=a, c=c: (a, c, i, 0))
    dblk = lambda a: pl.BlockSpec((1, _BN, 16), lambda i, a=a: (a, i, 0))
    w1blk = lambda r: pl.BlockSpec((_H, _D), lambda i, r=r: (r, 0))
    return pl.pallas_call(
        _mlp_body,
        grid=(_N // _BN,),
        in_specs=[
            pl.BlockSpec((_BN, _D), lambda i: (i, 0)),
            sblk(0, 0), sblk(0, 1), sblk(1, 0), sblk(1, 1),
            dblk(0), dblk(1),
            pl.BlockSpec((_D, _D), lambda i: (0, 0)),
            w1blk(2), w1blk(3), w1blk(4), w1blk(5),
            pl.BlockSpec((1, _D), lambda i: (0, 0)),
            pl.BlockSpec((_D, _EMB), lambda i: (0, 0)),
            pl.BlockSpec((1, _EMB), lambda i: (0, 0)),
        ],
        out_specs=pl.BlockSpec((_BN, _EMB), lambda i: (i, 0)),
        out_shape=jax.ShapeDtypeStruct((_N, _EMB), jnp.float32),
    )(x, sums, sums, sums, sums, degw, degw, W1, W1, W1, W1, W1,
      b1.reshape(1, _D), W2, b2.reshape(1, _EMB))


def kernel(x, edge_indices, W1, b1, W2, b2):
    x2 = x.reshape(2 * _N, _H)
    er = edge_indices.reshape(_NADJ, 2, _NS, _P, _PC, _C)
    sums, degw = _agg(x2, er)
    sums = sums.reshape(_NADJ, _NC, _N, _H)
    degw = degw.reshape(_NADJ, _N, 16)
    return _mlp(x, sums, degw, W1, b1, W2, b2)


# speedup vs baseline: 5.5615x; 1.1060x over previous
import functools

import jax
import jax.numpy as jnp
from jax import lax
from jax.experimental import pallas as pl
from jax.experimental.pallas import tpu as pltpu
from jax.experimental.pallas import tpu_sc as plsc

_N = 10000
_E = 160000
_D = 256
_EMB = 512
_NADJ = 2
_NC = 2
_NS = 16
_H = 128
_ET = _E // _NS
_C = 80
_CH = _ET // _C
_P = 5
_PC = _CH // _P
_RT = _N // _NS
_ZR = 25
_ZD = 125

_sc_mesh = plsc.VectorSubcoreMesh(
    core_axis_name="c", subcore_axis_name="s", num_cores=_NC, num_subcores=_NS
)


@functools.partial(
    pl.kernel,
    out_type=(
        jax.ShapeDtypeStruct((_NADJ, _NC, _NS, _RT, _H), jnp.float32),
        jax.ShapeDtypeStruct((_NADJ, _NS, _RT, 16), jnp.float32),
    ),
    mesh=_sc_mesh,
    scratch_types=[
        pltpu.VMEM_SHARED((_N, _H), jnp.float32),
        pltpu.VMEM_SHARED((_N, 16), jnp.float32),
        pltpu.VMEM((_PC, _C), jnp.int32),
        pltpu.VMEM((_PC, _C), jnp.int32),
        pltpu.VMEM((_C, _H), jnp.float32),
        pltpu.VMEM((_C, _H), jnp.float32),
        pltpu.VMEM((_C, 16), jnp.float32),
        pltpu.VMEM((_ZR, _H), jnp.float32),
        pltpu.VMEM((_ZD, 16), jnp.float32),
        pltpu.SemaphoreType.DMA,
        pltpu.SemaphoreType.DMA,
        pltpu.SemaphoreType.DMA,
    ],
    compiler_params=pltpu.CompilerParams(use_tc_tiling_on_sc=False),
)
def _agg(x2_hbm, er_hbm, sums_hbm, degw_hbm, acc_sh, deg_sh, src_v, dst_v,
         rows0_v, rows1_v, ones_v, zrow_v, zdeg_v, semg0, semg1, semd):
    c = lax.axis_index("c")
    s = lax.axis_index("s")
    row0 = s * _RT

    ones16 = jnp.ones((16,), jnp.float32)
    zeros16 = jnp.zeros((16,), jnp.float32)

    def init_ones(i, carry):
        ones_v[i, :] = ones16
        return carry

    lax.fori_loop(0, _C, init_ones, 0)

    def init_zrow(i, carry):
        for t in range(_H // 16):
            zrow_v[i, pl.ds(t * 16, 16)] = zeros16
        return carry

    lax.fori_loop(0, _ZR, init_zrow, 0)

    def init_zdeg(i, carry):
        zdeg_v[i, :] = zeros16
        return carry

    lax.fori_loop(0, _ZD, init_zdeg, 0)

    def g_start(j, buf, sem):
        pltpu.async_copy(x2_hbm.at[src_v.at[j]], buf, sem)

    def g_wait(buf, sem):
        pltpu.make_async_copy(x2_hbm.at[src_v.at[0]], buf, sem).wait()

    def stage(a, p):
        pltpu.sync_copy(er_hbm.at[a, 0, s, p], src_v)
        pltpu.sync_copy(er_hbm.at[a, 1, s, p], dst_v)

        def adjust(ch, carry):
            for t in range(_C // 16):
                sl = pl.ds(t * 16, 16)
                v = src_v[ch, sl]
                src_v[ch, sl] = v + v + c
            return carry

        lax.fori_loop(0, _PC, adjust, 0)

    def run_pass(a):
        do_deg = c == a

        def scatter(j, buf):
            @pl.when(do_deg)
            def _():
                pltpu.async_copy(ones_v, deg_sh.at[dst_v.at[j]], semd, add=True)

            pltpu.sync_copy(buf, acc_sh.at[dst_v.at[j]], add=True)

            @pl.when(do_deg)
            def _():
                pltpu.make_async_copy(ones_v, deg_sh.at[dst_v.at[0]], semd).wait()

        g_start(0, rows0_v, semg0)

        def pair(k, carry):
            j0 = 2 * k
            g_wait(rows0_v, semg0)
            g_start(j0 + 1, rows1_v, semg1)
            scatter(j0, rows0_v)
            g_wait(rows1_v, semg1)
            g_start(j0 + 2, rows0_v, semg0)
            scatter(j0 + 1, rows1_v)
            return carry

        lax.fori_loop(0, (_PC - 1) // 2, pair, 0)
        g_wait(rows0_v, semg0)
        scatter(_PC - 1, rows0_v)

    for a in range(_NADJ):
        def zero_acc(z, carry):
            pltpu.sync_copy(zrow_v, acc_sh.at[pl.ds(row0 + z * _ZR, _ZR)])
            return carry

        lax.fori_loop(0, _RT // _ZR, zero_acc, 0)
        for z in range(_RT // _ZD):
            pltpu.sync_copy(zdeg_v, deg_sh.at[pl.ds(row0 + z * _ZD, _ZD)])
        stage(a, 0)
        plsc.subcore_barrier()

        run_pass(a)
        for p in range(1, _P):
            stage(a, p)
            run_pass(a)
        plsc.subcore_barrier()

        pltpu.sync_copy(acc_sh.at[pl.ds(row0, _RT)], sums_hbm.at[a, c, s])

        @pl.when(c == a)
        def _():
            pltpu.sync_copy(deg_sh.at[pl.ds(row0, _RT)], degw_hbm.at[a, s])


_BN = 2000


def _split_bf16(v):
    hi = v.astype(jnp.bfloat16)
    lo = (v - hi.astype(jnp.float32)).astype(jnp.bfloat16)
    return hi, lo


def _dot3(a, bhi, blo):
    ahi, alo = _split_bf16(a)
    d = functools.partial(jnp.dot, preferred_element_type=jnp.float32)
    return d(ahi, bhi) + (d(ahi, blo) + d(alo, bhi))


def _mlp_body(x_ref, s00_ref, s01_ref, s10_ref, s11_ref, d0_ref, d1_ref,
              w1xh_ref, w1ah_ref, w1bh_ref, w1ch_ref, w1dh_ref,
              w1xl_ref, w1al_ref, w1bl_ref, w1cl_ref, w1dl_ref, b1_ref,
              w2h_ref, w2l_ref, b2_ref, out_ref):
    r0 = 1.0 / jnp.maximum(d0_ref[0, :, 0:1], 1.0)
    r1 = 1.0 / jnp.maximum(d1_ref[0, :, 0:1], 1.0)
    acc = _dot3(x_ref[...], w1xh_ref[...], w1xl_ref[...])
    acc = acc + _dot3(s00_ref[0, 0] * r0, w1ah_ref[...], w1al_ref[...])
    acc = acc + _dot3(s01_ref[0, 0] * r0, w1bh_ref[...], w1bl_ref[...])
    acc = acc + _dot3(s10_ref[0, 0] * r1, w1ch_ref[...], w1cl_ref[...])
    acc = acc + _dot3(s11_ref[0, 0] * r1, w1dh_ref[...], w1dl_ref[...])
    h = jnp.tanh(acc + b1_ref[...])
    out_ref[...] = _dot3(h, w2h_ref[...], w2l_ref[...]) + b2_ref[...]


def _mlp(x, sums, degw, W1, b1, W2, b2):
    w1hi, w1lo = _split_bf16(W1)
    w2hi, w2lo = _split_bf16(W2)
    sblk = lambda a, c: pl.BlockSpec((1, 1, _BN, _H), lambda i, a=a, c=c: (a, c, i, 0))
    dblk = lambda a: pl.BlockSpec((1, _BN, 16), lambda i, a=a: (a, i, 0))
    w1x_spec = pl.BlockSpec((_D, _D), lambda i: (0, 0))
    w1blk = lambda r: pl.BlockSpec((_H, _D), lambda i, r=r: (r, 0))
    w1specs = [w1x_spec, w1blk(2), w1blk(3), w1blk(4), w1blk(5)]
    return pl.pallas_call(
        _mlp_body,
        grid=(_N // _BN,),
        in_specs=[
            pl.BlockSpec((_BN, _D), lambda i: (i, 0)),
            sblk(0, 0), sblk(0, 1), sblk(1, 0), sblk(1, 1),
            dblk(0), dblk(1),
            *w1specs,
            *w1specs,
            pl.BlockSpec((1, _D), lambda i: (0, 0)),
            pl.BlockSpec((_D, _EMB), lambda i: (0, 0)),
            pl.BlockSpec((_D, _EMB), lambda i: (0, 0)),
            pl.BlockSpec((1, _EMB), lambda i: (0, 0)),
        ],
        out_specs=pl.BlockSpec((_BN, _EMB), lambda i: (i, 0)),
        out_shape=jax.ShapeDtypeStruct((_N, _EMB), jnp.float32),
    )(x, sums, sums, sums, sums, degw, degw,
      w1hi, w1hi, w1hi, w1hi, w1hi, w1lo, w1lo, w1lo, w1lo, w1lo,
      b1.reshape(1, _D), w2hi, w2lo, b2.reshape(1, _EMB))


def kernel(x, edge_indices, W1, b1, W2, b2):
    x2 = x.reshape(2 * _N, _H)
    er = edge_indices.reshape(_NADJ, 2, _NS, _P, _PC, _C)
    sums, degw = _agg(x2, er)
    sums = sums.reshape(_NADJ, _NC, _N, _H)
    degw = degw.reshape(_NADJ, _N, 16)
    return _mlp(x, sums, degw, W1, b1, W2, b2)


# speedup vs baseline: 5.6856x; 1.0223x over previous
import functools

import jax
import jax.numpy as jnp
from jax import lax
from jax.experimental import pallas as pl
from jax.experimental.pallas import tpu as pltpu
from jax.experimental.pallas import tpu_sc as plsc

_N = 10000
_E = 160000
_D = 256
_EMB = 512
_NADJ = 2
_NC = 2
_NS = 16
_H = 128
_ET = _E // _NS
_C = 80
_CH = _ET // _C
_P = 5
_PC = _CH // _P
_RT = _N // _NS
_ZR = 25
_ZD = 125

_sc_mesh = plsc.VectorSubcoreMesh(
    core_axis_name="c", subcore_axis_name="s", num_cores=_NC, num_subcores=_NS
)


@functools.partial(
    pl.kernel,
    out_type=(
        jax.ShapeDtypeStruct((_NADJ, _NC, _NS, _RT, _H), jnp.float32),
        jax.ShapeDtypeStruct((_NADJ, _NS, _RT, 16), jnp.float32),
    ),
    mesh=_sc_mesh,
    scratch_types=[
        pltpu.VMEM_SHARED((_N, _H), jnp.float32),
        pltpu.VMEM_SHARED((_N, 16), jnp.float32),
        pltpu.VMEM((2, _PC, _C), jnp.int32),
        pltpu.VMEM((2, _PC, _C), jnp.int32),
        pltpu.VMEM((_C, _H), jnp.float32),
        pltpu.VMEM((_C, _H), jnp.float32),
        pltpu.VMEM((_C, 16), jnp.float32),
        pltpu.VMEM((_ZR, _H), jnp.float32),
        pltpu.VMEM((_ZD, 16), jnp.float32),
        pltpu.SemaphoreType.DMA,
        pltpu.SemaphoreType.DMA,
        pltpu.SemaphoreType.DMA,
        pltpu.SemaphoreType.DMA,
        pltpu.SemaphoreType.DMA,
    ],
    compiler_params=pltpu.CompilerParams(use_tc_tiling_on_sc=False),
)
def _agg(x2_hbm, es_hbm, ed_hbm, sums_hbm, degw_hbm, acc_sh, deg_sh, src_v,
         dst_v, rows0_v, rows1_v, ones_v, zrow_v, zdeg_v, semg0, semg1, semd,
         seme, semz):
    c = lax.axis_index("c")
    s = lax.axis_index("s")
    row0 = s * _RT

    ones16 = jnp.ones((16,), jnp.float32)
    zeros16 = jnp.zeros((16,), jnp.float32)

    def init_ones(i, carry):
        ones_v[i, :] = ones16
        return carry

    lax.fori_loop(0, _C, init_ones, 0)

    def init_zrow(i, carry):
        for t in range(_H // 16):
            zrow_v[i, pl.ds(t * 16, 16)] = zeros16
        return carry

    lax.fori_loop(0, _ZR, init_zrow, 0)

    def init_zdeg(i, carry):
        zdeg_v[i, :] = zeros16
        return carry

    lax.fori_loop(0, _ZD, init_zdeg, 0)

    def g_start(b, j, buf, sem):
        pltpu.async_copy(x2_hbm.at[src_v.at[b, j]], buf, sem)

    def g_wait(buf, sem):
        pltpu.make_async_copy(x2_hbm.at[src_v.at[0, 0]], buf, sem).wait()

    def stage_start(a, p, b):
        pltpu.async_copy(es_hbm.at[c, a, s, p], src_v.at[b], seme)
        pltpu.async_copy(ed_hbm.at[a, s, p], dst_v.at[b], seme)

    def stage_wait(b):
        pltpu.make_async_copy(es_hbm.at[0, 0, 0, 0], src_v.at[b], seme).wait()
        pltpu.make_async_copy(ed_hbm.at[0, 0, 0], dst_v.at[b], seme).wait()

    def run_pass(a, b):
        do_deg = c == a

        def scatter(j, buf):
            @pl.when(do_deg)
            def _():
                pltpu.async_copy(ones_v, deg_sh.at[dst_v.at[b, j]], semd,
                                 add=True)

            pltpu.sync_copy(buf, acc_sh.at[dst_v.at[b, j]], add=True)

            @pl.when(do_deg)
            def _():
                pltpu.make_async_copy(ones_v, deg_sh.at[dst_v.at[0, 0]],
                                      semd).wait()

        g_start(b, 0, rows0_v, semg0)

        def pair(k, carry):
            j0 = 2 * k
            g_wait(rows0_v, semg0)
            g_start(b, j0 + 1, rows1_v, semg1)
            scatter(j0, rows0_v)
            g_wait(rows1_v, semg1)
            g_start(b, j0 + 2, rows0_v, semg0)
            scatter(j0 + 1, rows1_v)
            return carry

        lax.fori_loop(0, (_PC - 1) // 2, pair, 0)
        g_wait(rows0_v, semg0)
        scatter(_PC - 1, rows0_v)

    for a in range(_NADJ):
        stage_start(a, 0, 0)
        def zero_acc(z, carry):
            pltpu.async_copy(zrow_v, acc_sh.at[pl.ds(row0 + z * _ZR, _ZR)],
                             semz)
            return carry

        lax.fori_loop(0, _RT // _ZR, zero_acc, 0)
        for z in range(_RT // _ZD):
            pltpu.async_copy(zdeg_v, deg_sh.at[pl.ds(row0 + z * _ZD, _ZD)],
                             semz)

        def zero_drain(z, carry):
            pltpu.make_async_copy(zrow_v, acc_sh.at[pl.ds(row0, _ZR)],
                                  semz).wait()
            return carry

        lax.fori_loop(0, _RT // _ZR, zero_drain, 0)
        for z in range(_RT // _ZD):
            pltpu.make_async_copy(zdeg_v, deg_sh.at[pl.ds(row0, _ZD)],
                                  semz).wait()
        stage_wait(0)
        plsc.subcore_barrier()

        for p in range(_P):
            b = p % 2
            if p + 1 < _P:
                stage_start(a, p + 1, 1 - b)
            run_pass(a, b)
            if p + 1 < _P:
                stage_wait(1 - b)
        plsc.subcore_barrier()

        pltpu.sync_copy(acc_sh.at[pl.ds(row0, _RT)], sums_hbm.at[a, c, s])

        @pl.when(c == a)
        def _():
            pltpu.sync_copy(deg_sh.at[pl.ds(row0, _RT)], degw_hbm.at[a, s])


_BN = 2000


def _split_bf16(v):
    hi = v.astype(jnp.bfloat16)
    lo = (v - hi.astype(jnp.float32)).astype(jnp.bfloat16)
    return hi, lo


def _dot3(a, bhi, blo):
    ahi, alo = _split_bf16(a)
    d = functools.partial(jnp.dot, preferred_element_type=jnp.float32)
    return d(ahi, bhi) + (d(ahi, blo) + d(alo, bhi))


def _mlp_body(x_ref, s00_ref, s01_ref, s10_ref, s11_ref, d0_ref, d1_ref,
              w1xh_ref, w1ah_ref, w1bh_ref, w1ch_ref, w1dh_ref,
              w1xl_ref, w1al_ref, w1bl_ref, w1cl_ref, w1dl_ref, b1_ref,
              w2h_ref, w2l_ref, b2_ref, out_ref):
    r0 = 1.0 / jnp.maximum(d0_ref[0, :, 0:1], 1.0)
    r1 = 1.0 / jnp.maximum(d1_ref[0, :, 0:1], 1.0)
    acc = _dot3(x_ref[...], w1xh_ref[...], w1xl_ref[...])
    acc = acc + _dot3(s00_ref[0, 0] * r0, w1ah_ref[...], w1al_ref[...])
    acc = acc + _dot3(s01_ref[0, 0] * r0, w1bh_ref[...], w1bl_ref[...])
    acc = acc + _dot3(s10_ref[0, 0] * r1, w1ch_ref[...], w1cl_ref[...])
    acc = acc + _dot3(s11_ref[0, 0] * r1, w1dh_ref[...], w1dl_ref[...])
    h = jnp.tanh(acc + b1_ref[...])
    out_ref[...] = _dot3(h, w2h_ref[...], w2l_ref[...]) + b2_ref[...]


def _mlp(x, sums, degw, W1, b1, W2, b2):
    w1hi, w1lo = _split_bf16(W1)
    w2hi, w2lo = _split_bf16(W2)
    sblk = lambda a, c: pl.BlockSpec((1, 1, _BN, _H), lambda i, a=a, c=c: (a, c, i, 0))
    dblk = lambda a: pl.BlockSpec((1, _BN, 16), lambda i, a=a: (a, i, 0))
    w1x_spec = pl.BlockSpec((_D, _D), lambda i: (0, 0))
    w1blk = lambda r: pl.BlockSpec((_H, _D), lambda i, r=r: (r, 0))
    w1specs = [w1x_spec, w1blk(2), w1blk(3), w1blk(4), w1blk(5)]
    return pl.pallas_call(
        _mlp_body,
        grid=(_N // _BN,),
        in_specs=[
            pl.BlockSpec((_BN, _D), lambda i: (i, 0)),
            sblk(0, 0), sblk(0, 1), sblk(1, 0), sblk(1, 1),
            dblk(0), dblk(1),
            *w1specs,
            *w1specs,
            pl.BlockSpec((1, _D), lambda i: (0, 0)),
            pl.BlockSpec((_D, _EMB), lambda i: (0, 0)),
            pl.BlockSpec((_D, _EMB), lambda i: (0, 0)),
            pl.BlockSpec((1, _EMB), lambda i: (0, 0)),
        ],
        out_specs=pl.BlockSpec((_BN, _EMB), lambda i: (i, 0)),
        out_shape=jax.ShapeDtypeStruct((_N, _EMB), jnp.float32),
    )(x, sums, sums, sums, sums, degw, degw,
      w1hi, w1hi, w1hi, w1hi, w1hi, w1lo, w1lo, w1lo, w1lo, w1lo,
      b1.reshape(1, _D), w2hi, w2lo, b2.reshape(1, _EMB))


def kernel(x, edge_indices, W1, b1, W2, b2):
    x2 = x.reshape(2 * _N, _H)
    src2 = edge_indices[:, 0] * 2
    es = jnp.stack([src2, src2 + 1]).reshape(_NC, _NADJ, _NS, _P, _PC, _C)
    ed = edge_indices[:, 1].reshape(_NADJ, _NS, _P, _PC, _C)
    sums, degw = _agg(x2, es, ed)
    sums = sums.reshape(_NADJ, _NC, _N, _H)
    degw = degw.reshape(_NADJ, _N, 16)
    return _mlp(x, sums, degw, W1, b1, W2, b2)


# speedup vs baseline: 6.1176x; 1.0760x over previous
import functools

import jax
import jax.numpy as jnp
from jax import lax
from jax.experimental import pallas as pl
from jax.experimental.pallas import tpu as pltpu
from jax.experimental.pallas import tpu_sc as plsc

_N = 10000
_E = 160000
_D = 256
_EMB = 512
_NADJ = 2
_NC = 2
_NS = 16
_H = 128
_ET = _E // _NS
_C = 100
_CH = _ET // _C
_P = 5
_PC = _CH // _P
_RT = _N // _NS
_ZR = 25
_ZD = 125

_sc_mesh = plsc.VectorSubcoreMesh(
    core_axis_name="c", subcore_axis_name="s", num_cores=_NC, num_subcores=_NS
)


@functools.partial(
    pl.kernel,
    out_type=(
        jax.ShapeDtypeStruct((_NADJ, _NC, _NS, _RT, _H), jnp.float32),
        jax.ShapeDtypeStruct((_NADJ, _NS, _RT, 16), jnp.float32),
    ),
    mesh=_sc_mesh,
    scratch_types=[
        pltpu.VMEM_SHARED((_N, _H), jnp.float32),
        pltpu.VMEM_SHARED((_N, 16), jnp.float32),
        pltpu.VMEM((2, _PC, _C), jnp.int32),
        pltpu.VMEM((2, _PC, _C), jnp.int32),
        pltpu.VMEM((_C, _H), jnp.float32),
        pltpu.VMEM((_C, _H), jnp.float32),
        pltpu.VMEM((_C, 16), jnp.float32),
        pltpu.VMEM((_ZR, _H), jnp.float32),
        pltpu.VMEM((_ZD, 16), jnp.float32),
        pltpu.SemaphoreType.DMA,
        pltpu.SemaphoreType.DMA,
        pltpu.SemaphoreType.DMA,
        pltpu.SemaphoreType.DMA,
        pltpu.SemaphoreType.DMA,
    ],
    compiler_params=pltpu.CompilerParams(use_tc_tiling_on_sc=False),
)
def _agg(x2_hbm, es_hbm, ed_hbm, sums_hbm, degw_hbm, acc_sh, deg_sh, src_v,
         dst_v, rows0_v, rows1_v, ones_v, zrow_v, zdeg_v, semg0, semg1, semd,
         seme, semz):
    c = lax.axis_index("c")
    s = lax.axis_index("s")
    row0 = s * _RT

    ones16 = jnp.ones((16,), jnp.float32)
    zeros16 = jnp.zeros((16,), jnp.float32)

    def init_ones(i, carry):
        ones_v[i, :] = ones16
        return carry

    lax.fori_loop(0, _C, init_ones, 0)

    def init_zrow(i, carry):
        for t in range(_H // 16):
            zrow_v[i, pl.ds(t * 16, 16)] = zeros16
        return carry

    lax.fori_loop(0, _ZR, init_zrow, 0)

    def init_zdeg(i, carry):
        zdeg_v[i, :] = zeros16
        return carry

    lax.fori_loop(0, _ZD, init_zdeg, 0)

    def g_start(b, j, buf, sem):
        pltpu.async_copy(x2_hbm.at[src_v.at[b, j]], buf, sem)

    def g_wait(buf, sem):
        pltpu.make_async_copy(x2_hbm.at[src_v.at[0, 0]], buf, sem).wait()

    def stage_start(a, p, b):
        pltpu.async_copy(es_hbm.at[c, a, s, p], src_v.at[b], seme)
        pltpu.async_copy(ed_hbm.at[a, s, p], dst_v.at[b], seme)

    def stage_wait(b):
        pltpu.make_async_copy(es_hbm.at[0, 0, 0, 0], src_v.at[b], seme).wait()
        pltpu.make_async_copy(ed_hbm.at[0, 0, 0], dst_v.at[b], seme).wait()

    def run_pass(a, b):
        do_deg = c == a

        def scatter(j, buf):
            @pl.when(do_deg)
            def _():
                pltpu.async_copy(ones_v, deg_sh.at[dst_v.at[b, j]], semd,
                                 add=True)

            pltpu.sync_copy(buf, acc_sh.at[dst_v.at[b, j]], add=True)

            @pl.when(do_deg)
            def _():
                pltpu.make_async_copy(ones_v, deg_sh.at[dst_v.at[0, 0]],
                                      semd).wait()

        g_start(b, 0, rows0_v, semg0)

        def pair(k, carry):
            j0 = 2 * k
            g_wait(rows0_v, semg0)
            g_start(b, j0 + 1, rows1_v, semg1)
            scatter(j0, rows0_v)
            g_wait(rows1_v, semg1)
            g_start(b, j0 + 2, rows0_v, semg0)
            scatter(j0 + 1, rows1_v)
            return carry

        lax.fori_loop(0, (_PC - 1) // 2, pair, 0)
        if _PC % 2:
            g_wait(rows0_v, semg0)
            scatter(_PC - 1, rows0_v)
        else:
            g_wait(rows0_v, semg0)
            g_start(b, _PC - 1, rows1_v, semg1)
            scatter(_PC - 2, rows0_v)
            g_wait(rows1_v, semg1)
            scatter(_PC - 1, rows1_v)

    for a in range(_NADJ):
        stage_start(a, 0, 0)
        def zero_acc(z, carry):
            pltpu.async_copy(zrow_v, acc_sh.at[pl.ds(row0 + z * _ZR, _ZR)],
                             semz)
            return carry

        lax.fori_loop(0, _RT // _ZR, zero_acc, 0)
        for z in range(_RT // _ZD):
            pltpu.async_copy(zdeg_v, deg_sh.at[pl.ds(row0 + z * _ZD, _ZD)],
                             semz)

        def zero_drain(z, carry):
            pltpu.make_async_copy(zrow_v, acc_sh.at[pl.ds(row0, _ZR)],
                                  semz).wait()
            return carry

        lax.fori_loop(0, _RT // _ZR, zero_drain, 0)
        for z in range(_RT // _ZD):
            pltpu.make_async_copy(zdeg_v, deg_sh.at[pl.ds(row0, _ZD)],
                                  semz).wait()
        stage_wait(0)
        plsc.subcore_barrier()

        for p in range(_P):
            b = p % 2
            if p + 1 < _P:
                stage_start(a, p + 1, 1 - b)
            run_pass(a, b)
            if p + 1 < _P:
                stage_wait(1 - b)
        plsc.subcore_barrier()

        pltpu.sync_copy(acc_sh.at[pl.ds(row0, _RT)], sums_hbm.at[a, c, s])

        @pl.when(c == a)
        def _():
            pltpu.sync_copy(deg_sh.at[pl.ds(row0, _RT)], degw_hbm.at[a, s])


_BN = 2000


def _split_bf16(v):
    hi = v.astype(jnp.bfloat16)
    lo = (v - hi.astype(jnp.float32)).astype(jnp.bfloat16)
    return hi, lo


def _dot3(a, bhi, blo):
    ahi, alo = _split_bf16(a)
    d = functools.partial(jnp.dot, preferred_element_type=jnp.float32)
    return d(ahi, bhi) + (d(ahi, blo) + d(alo, bhi))


def _mlp_body(x_ref, s00_ref, s01_ref, s10_ref, s11_ref, d0_ref, d1_ref,
              w1xh_ref, w1ah_ref, w1bh_ref, w1ch_ref, w1dh_ref,
              w1xl_ref, w1al_ref, w1bl_ref, w1cl_ref, w1dl_ref, b1_ref,
              w2h_ref, w2l_ref, b2_ref, out_ref):
    r0 = 1.0 / jnp.maximum(d0_ref[0, :, 0:1], 1.0)
    r1 = 1.0 / jnp.maximum(d1_ref[0, :, 0:1], 1.0)
    acc = _dot3(x_ref[...], w1xh_ref[...], w1xl_ref[...])
    acc = acc + _dot3(s00_ref[0, 0] * r0, w1ah_ref[...], w1al_ref[...])
    acc = acc + _dot3(s01_ref[0, 0] * r0, w1bh_ref[...], w1bl_ref[...])
    acc = acc + _dot3(s10_ref[0, 0] * r1, w1ch_ref[...], w1cl_ref[...])
    acc = acc + _dot3(s11_ref[0, 0] * r1, w1dh_ref[...], w1dl_ref[...])
    h = jnp.tanh(acc + b1_ref[...])
    out_ref[...] = _dot3(h, w2h_ref[...], w2l_ref[...]) + b2_ref[...]


def _mlp(x, sums, degw, W1, b1, W2, b2):
    w1hi, w1lo = _split_bf16(W1)
    w2hi, w2lo = _split_bf16(W2)
    sblk = lambda a, c: pl.BlockSpec((1, 1, _BN, _H), lambda i, a=a, c=c: (a, c, i, 0))
    dblk = lambda a: pl.BlockSpec((1, _BN, 16), lambda i, a=a: (a, i, 0))
    w1x_spec = pl.BlockSpec((_D, _D), lambda i: (0, 0))
    w1blk = lambda r: pl.BlockSpec((_H, _D), lambda i, r=r: (r, 0))
    w1specs = [w1x_spec, w1blk(2), w1blk(3), w1blk(4), w1blk(5)]
    return pl.pallas_call(
        _mlp_body,
        grid=(_N // _BN,),
        in_specs=[
            pl.BlockSpec((_BN, _D), lambda i: (i, 0)),
            sblk(0, 0), sblk(0, 1), sblk(1, 0), sblk(1, 1),
            dblk(0), dblk(1),
            *w1specs,
            *w1specs,
            pl.BlockSpec((1, _D), lambda i: (0, 0)),
            pl.BlockSpec((_D, _EMB), lambda i: (0, 0)),
            pl.BlockSpec((_D, _EMB), lambda i: (0, 0)),
            pl.BlockSpec((1, _EMB), lambda i: (0, 0)),
        ],
        out_specs=pl.BlockSpec((_BN, _EMB), lambda i: (i, 0)),
        out_shape=jax.ShapeDtypeStruct((_N, _EMB), jnp.float32),
    )(x, sums, sums, sums, sums, degw, degw,
      w1hi, w1hi, w1hi, w1hi, w1hi, w1lo, w1lo, w1lo, w1lo, w1lo,
      b1.reshape(1, _D), w2hi, w2lo, b2.reshape(1, _EMB))


def kernel(x, edge_indices, W1, b1, W2, b2):
    x2 = x.reshape(2 * _N, _H)
    src2 = edge_indices[:, 0] * 2
    es = jnp.stack([src2, src2 + 1]).reshape(_NC, _NADJ, _NS, _P, _PC, _C)
    ed = edge_indices[:, 1].reshape(_NADJ, _NS, _P, _PC, _C)
    sums, degw = _agg(x2, es, ed)
    sums = sums.reshape(_NADJ, _NC, _N, _H)
    degw = degw.reshape(_NADJ, _N, 16)
    return _mlp(x, sums, degw, W1, b1, W2, b2)
